# Initial kernel scaffold; baseline (speedup 1.0000x reference)
import jax
import jax.numpy as jnp
from jax.experimental import pallas as pl


def kernel(embedding_features_per_residue, edge_index, batch, embedding_features_per_sequence, W1, b1, g1, be1, W2, b2, g2, be2, W3, b3, g3, be3, Wfc, bfc, Wlin, blin):
    raise NotImplementedError("write your pallas kernel here")


# speedup vs baseline: 11.0544x; 11.0544x over previous
import functools

import jax
import jax.numpy as jnp
from jax import lax
from jax.experimental import pallas as pl
from jax.experimental.pallas import tpu as pltpu
from jax.experimental.pallas import tpu_sc as plsc

N = 10000
E = 320000
NPAD = 10240
SLAB = NPAD // 16
NTILES = 32
ET = E // NTILES
CH = 80
NCH = ET // CH


def _deg_body(dst_hbm, zvec_hbm, out_hbm, didx, ones_v, deg_sh):
  c = lax.axis_index("c")
  s = lax.axis_index("s")
  tid = s * 2 + c
  base = tid * ET

  pltpu.sync_copy(zvec_hbm.at[pl.ds(s * SLAB, SLAB)],
                  deg_sh.at[pl.ds(s * SLAB, SLAB)])
  for i in range(CH // 16):
    ones_v[pl.ds(i * 16, 16)] = jnp.ones((16,), jnp.float32)
  plsc.subcore_barrier()

  def step(k, carry):
    pltpu.sync_copy(dst_hbm.at[pl.ds(base + k * CH, CH)], didx)
    pltpu.sync_copy(ones_v, deg_sh.at[didx], add=True)
    return carry

  lax.fori_loop(0, NCH, step, 0)
  plsc.subcore_barrier()
  pltpu.sync_copy(deg_sh.at[pl.ds(s * SLAB, SLAB)],
                  out_hbm.at[c, pl.ds(s * SLAB, SLAB)])


def _scatter_body(src_hbm, dst_hbm, hp_hbm, zrows_hbm, out_hbm,
                  sidx, didx, rows, acc_sh, gsem):
  c = lax.axis_index("c")
  s = lax.axis_index("s")
  tid = s * 2 + c
  base = tid * ET

  pltpu.sync_copy(zrows_hbm.at[pl.ds(s * SLAB, SLAB)],
                  acc_sh.at[pl.ds(s * SLAB, SLAB)])
  plsc.subcore_barrier()

  def step(k, carry):
    pltpu.sync_copy(src_hbm.at[pl.ds(base + k * CH, CH)], sidx)
    pltpu.async_copy(hp_hbm.at[sidx], rows, gsem).wait()
    pltpu.sync_copy(dst_hbm.at[pl.ds(base + k * CH, CH)], didx)
    pltpu.sync_copy(rows, acc_sh.at[didx], add=True)
    return carry

  lax.fori_loop(0, NCH, step, 0)
  plsc.subcore_barrier()
  pltpu.sync_copy(acc_sh.at[pl.ds(s * SLAB, SLAB)],
                  out_hbm.at[c, pl.ds(s * SLAB, SLAB)])


@functools.lru_cache(maxsize=None)
def _sc_calls():
  mesh = plsc.VectorSubcoreMesh(core_axis_name="c", subcore_axis_name="s")
  deg_call = pl.kernel(
      _deg_body,
      out_type=jax.ShapeDtypeStruct((2, NPAD), jnp.float32),
      mesh=mesh,
      scratch_types=[
          pltpu.VMEM((CH,), jnp.int32),
          pltpu.VMEM((CH,), jnp.float32),
          pltpu.VMEM_SHARED((NPAD,), jnp.float32),
      ],
  )
  scatter_call = pl.kernel(
      _scatter_body,
      out_type=jax.ShapeDtypeStruct((2, NPAD, 128), jnp.float32),
      mesh=mesh,
      scratch_types=[
          pltpu.VMEM((CH,), jnp.int32),
          pltpu.VMEM((CH,), jnp.int32),
          pltpu.VMEM((CH, 128), jnp.float32),
          pltpu.VMEM_SHARED((NPAD, 128), jnp.float32),
          pltpu.SemaphoreType.DMA,
      ],
  )
  return deg_call, scatter_call


def _dinv(degp_ref):
  deg = degp_ref[0] + degp_ref[1] + 1.0
  return lax.rsqrt(deg)


def _pre_body(degp_ref, x_ref, w_ref, hp_ref):
  h = jnp.dot(x_ref[...], w_ref[...], preferred_element_type=jnp.float32)
  hp_ref[...] = h * _dinv(degp_ref)


def _pre_call(degp, x, w):
  return pl.pallas_call(
      _pre_body,
      out_shape=jax.ShapeDtypeStruct((N, 128), jnp.float32),
  )(degp, x, w)


def _bn(z, g_ref, be_ref):
  mean = jnp.mean(z, axis=0, keepdims=True)
  d = z - mean
  var = jnp.mean(d * d, axis=0, keepdims=True)
  return d * lax.rsqrt(var + 1e-5) * g_ref[...] + be_ref[...]


def _mid_body(degp_ref, acc_ref, hp_ref, b_ref, g_ref, be_ref, w_ref,
              out_ref, *, relu):
  dinv = _dinv(degp_ref)
  z = dinv * (acc_ref[0] + acc_ref[1] + hp_ref[...]) + b_ref[...]
  if relu:
    z = jnp.maximum(z, 0.0)
  z = _bn(z, g_ref, be_ref)
  out_ref[...] = jnp.dot(z, w_ref[...],
                         preferred_element_type=jnp.float32) * dinv


def _mid_call(degp, acc, hp, b, g, be, w, relu):
  return pl.pallas_call(
      functools.partial(_mid_body, relu=relu),
      out_shape=jax.ShapeDtypeStruct((N, 128), jnp.float32),
  )(degp, acc, hp, b, g, be, w)


def _head_body(degp_ref, acc_ref, hp_ref, b_ref, g_ref, be_ref,
               batch_ref, seq_ref, wfc_ref, bfc_ref, wlin_ref, blin_ref,
               out_ref):
  dinv = _dinv(degp_ref)
  z = dinv * (acc_ref[0] + acc_ref[1] + hp_ref[...]) + b_ref[...]
  z = _bn(z, g_ref, be_ref)
  gids = lax.broadcasted_iota(jnp.int32, (16, N), 0)
  mask = (gids == batch_ref[...]).astype(jnp.float32)
  cnt = jnp.sum(mask, axis=1, keepdims=True)
  pooled = jnp.dot(mask, z, preferred_element_type=jnp.float32)
  pooled = pooled / jnp.maximum(cnt, 1.0)
  y = jnp.dot(seq_ref[...], wfc_ref[...],
              preferred_element_type=jnp.float32) + bfc_ref[...]
  o = jnp.dot(pooled + y, wlin_ref[...],
              preferred_element_type=jnp.float32) + blin_ref[...]
  out_ref[...] = jax.nn.sigmoid(o)


def _head_call(degp, acc, hp, b, g, be, batch, seq, wfc, bfc, wlin, blin):
  return pl.pallas_call(
      _head_body,
      out_shape=jax.ShapeDtypeStruct((16, 256), jnp.float32),
  )(degp, acc, hp, b, g, be, batch, seq, wfc, bfc, wlin, blin)


@jax.jit
def kernel(embedding_features_per_residue, edge_index, batch,
           embedding_features_per_sequence,
           W1, b1, g1, be1, W2, b2, g2, be2, W3, b3, g3, be3,
           Wfc, bfc, Wlin, blin):
  x = embedding_features_per_residue
  src = edge_index[0]
  dst = edge_index[1]
  zrows = jnp.zeros((NPAD, 128), jnp.float32)
  zvec = jnp.zeros((NPAD,), jnp.float32)

  deg_call, scatter_call = _sc_calls()
  degp = deg_call(dst, zvec)
  degp = degp[:, :N, None]

  b1r, g1r, be1r = b1[None], g1[None], be1[None]
  b2r, g2r, be2r = b2[None], g2[None], be2[None]
  b3r, g3r, be3r = b3[None], g3[None], be3[None]

  hp1 = _pre_call(degp, x, W1)
  acc1 = scatter_call(src, dst, hp1, zrows)[:, :N]
  hp2 = _mid_call(degp, acc1, hp1, b1r, g1r, be1r, W2, relu=True)
  acc2 = scatter_call(src, dst, hp2, zrows)[:, :N]
  hp3 = _mid_call(degp, acc2, hp2, b2r, g2r, be2r, W3, relu=True)
  acc3 = scatter_call(src, dst, hp3, zrows)[:, :N]
  return _head_call(degp, acc3, hp3, b3r, g3r, be3r, batch[None],
                    embedding_features_per_sequence, Wfc, bfc[None],
                    Wlin, blin[None])


# speedup vs baseline: 22.9048x; 2.0720x over previous
import functools

import jax
import jax.numpy as jnp
from jax import lax
from jax.experimental import pallas as pl
from jax.experimental.pallas import tpu as pltpu
from jax.experimental.pallas import tpu_sc as plsc

N = 10000
E = 320000
NPAD = 10240
SLAB = NPAD // 16
NTILES = 32
ET = E // NTILES
CH = 80
NCH = ET // CH


def _deg_body(dst_hbm, zvec_hbm, out_hbm, didx, ones_v, deg_sh):
  c = lax.axis_index("c")
  s = lax.axis_index("s")
  tid = s * 2 + c
  base = tid * ET

  pltpu.sync_copy(zvec_hbm.at[pl.ds(s * SLAB, SLAB)],
                  deg_sh.at[pl.ds(s * SLAB, SLAB)])
  for i in range(CH // 16):
    ones_v[pl.ds(i * 16, 16)] = jnp.ones((16,), jnp.float32)
  plsc.subcore_barrier()

  def step(k, carry):
    pltpu.sync_copy(dst_hbm.at[pl.ds(base + k * CH, CH)], didx)
    pltpu.sync_copy(ones_v, deg_sh.at[didx], add=True)
    return carry

  lax.fori_loop(0, NCH, step, 0)
  plsc.subcore_barrier()
  pltpu.sync_copy(deg_sh.at[pl.ds(s * SLAB, SLAB)],
                  out_hbm.at[c, pl.ds(s * SLAB, SLAB)])


def _scatter_body(src_hbm, dst_hbm, hp_hbm, zrows_hbm, out_hbm,
                  sidx_all, didx0, didx1, rows0, rows1, acc_sh,
                  gsem0, gsem1, dsem0, dsem1):
  c = lax.axis_index("c")
  s = lax.axis_index("s")
  tid = s * 2 + c
  base = tid * ET

  pltpu.sync_copy(zrows_hbm.at[pl.ds(s * SLAB, SLAB)],
                  acc_sh.at[pl.ds(s * SLAB, SLAB)])
  pltpu.sync_copy(src_hbm.at[pl.ds(base, ET)], sidx_all)
  plsc.subcore_barrier()

  didx = (didx0, didx1)
  rows = (rows0, rows1)
  gsem = (gsem0, gsem1)
  dsem = (dsem0, dsem1)

  def issue(i, b):
    pltpu.async_copy(dst_hbm.at[pl.ds(base + i * CH, CH)], didx[b], dsem[b])
    pltpu.async_copy(hp_hbm.at[sidx_all.at[pl.ds(i * CH, CH)]], rows[b],
                     gsem[b])

  for b in range(2):
    issue(b, b)

  def outer(j, carry):
    for b in range(2):
      i = j * 2 + b
      pltpu.make_async_copy(dst_hbm.at[pl.ds(base, CH)], didx[b],
                            dsem[b]).wait()
      pltpu.make_async_copy(hp_hbm.at[sidx_all.at[pl.ds(0, CH)]], rows[b],
                            gsem[b]).wait()
      pltpu.sync_copy(rows[b], acc_sh.at[didx[b]], add=True)

      @pl.when(i + 2 < NCH)
      def _():
        issue(i + 2, b)
    return carry

  lax.fori_loop(0, NCH // 2, outer, 0)
  if NCH % 2:
    b = (NCH - 1) % 2
    pltpu.make_async_copy(dst_hbm.at[pl.ds(base, CH)], didx[b],
                          dsem[b]).wait()
    pltpu.make_async_copy(hp_hbm.at[sidx_all.at[pl.ds(0, CH)]], rows[b],
                          gsem[b]).wait()
    pltpu.sync_copy(rows[b], acc_sh.at[didx[b]], add=True)
  plsc.subcore_barrier()
  pltpu.sync_copy(acc_sh.at[pl.ds(s * SLAB, SLAB)],
                  out_hbm.at[c, pl.ds(s * SLAB, SLAB)])


@functools.lru_cache(maxsize=None)
def _sc_calls():
  mesh = plsc.VectorSubcoreMesh(core_axis_name="c", subcore_axis_name="s")
  deg_call = pl.kernel(
      _deg_body,
      out_type=jax.ShapeDtypeStruct((2, NPAD), jnp.float32),
      mesh=mesh,
      scratch_types=[
          pltpu.VMEM((CH,), jnp.int32),
          pltpu.VMEM((CH,), jnp.float32),
          pltpu.VMEM_SHARED((NPAD,), jnp.float32),
      ],
  )
  scatter_call = pl.kernel(
      _scatter_body,
      out_type=jax.ShapeDtypeStruct((2, NPAD, 128), jnp.float32),
      mesh=mesh,
      scratch_types=[
          pltpu.VMEM((ET,), jnp.int32),
          pltpu.VMEM((CH,), jnp.int32),
          pltpu.VMEM((CH,), jnp.int32),
          pltpu.VMEM((CH, 128), jnp.float32),
          pltpu.VMEM((CH, 128), jnp.float32),
          pltpu.VMEM_SHARED((NPAD, 128), jnp.float32),
          pltpu.SemaphoreType.DMA,
          pltpu.SemaphoreType.DMA,
          pltpu.SemaphoreType.DMA,
          pltpu.SemaphoreType.DMA,
      ],
  )
  return deg_call, scatter_call


def _dinv(degp_ref):
  deg = degp_ref[0] + degp_ref[1] + 1.0
  return lax.rsqrt(deg)


def _pre_body(degp_ref, x_ref, w_ref, hp_ref):
  h = jnp.dot(x_ref[...], w_ref[...], preferred_element_type=jnp.float32)
  hp_ref[...] = h * _dinv(degp_ref)


def _pre_call(degp, x, w):
  return pl.pallas_call(
      _pre_body,
      out_shape=jax.ShapeDtypeStruct((N, 128), jnp.float32),
  )(degp, x, w)


def _bn(z, g_ref, be_ref):
  mean = jnp.mean(z, axis=0, keepdims=True)
  d = z - mean
  var = jnp.mean(d * d, axis=0, keepdims=True)
  return d * lax.rsqrt(var + 1e-5) * g_ref[...] + be_ref[...]


def _mid_body(degp_ref, acc_ref, hp_ref, b_ref, g_ref, be_ref, w_ref,
              out_ref, *, relu):
  dinv = _dinv(degp_ref)
  z = dinv * (acc_ref[0] + acc_ref[1] + hp_ref[...]) + b_ref[...]
  if relu:
    z = jnp.maximum(z, 0.0)
  z = _bn(z, g_ref, be_ref)
  out_ref[...] = jnp.dot(z, w_ref[...],
                         preferred_element_type=jnp.float32) * dinv


def _mid_call(degp, acc, hp, b, g, be, w, relu):
  return pl.pallas_call(
      functools.partial(_mid_body, relu=relu),
      out_shape=jax.ShapeDtypeStruct((N, 128), jnp.float32),
  )(degp, acc, hp, b, g, be, w)


def _head_body(degp_ref, acc_ref, hp_ref, b_ref, g_ref, be_ref,
               batch_ref, seq_ref, wfc_ref, bfc_ref, wlin_ref, blin_ref,
               out_ref):
  dinv = _dinv(degp_ref)
  z = dinv * (acc_ref[0] + acc_ref[1] + hp_ref[...]) + b_ref[...]
  z = _bn(z, g_ref, be_ref)
  gids = lax.broadcasted_iota(jnp.int32, (16, N), 0)
  mask = (gids == batch_ref[...]).astype(jnp.float32)
  cnt = jnp.sum(mask, axis=1, keepdims=True)
  pooled = jnp.dot(mask, z, preferred_element_type=jnp.float32)
  pooled = pooled / jnp.maximum(cnt, 1.0)
  y = jnp.dot(seq_ref[...], wfc_ref[...],
              preferred_element_type=jnp.float32) + bfc_ref[...]
  o = jnp.dot(pooled + y, wlin_ref[...],
              preferred_element_type=jnp.float32) + blin_ref[...]
  out_ref[...] = jax.nn.sigmoid(o)


def _head_call(degp, acc, hp, b, g, be, batch, seq, wfc, bfc, wlin, blin):
  return pl.pallas_call(
      _head_body,
      out_shape=jax.ShapeDtypeStruct((16, 256), jnp.float32),
  )(degp, acc, hp, b, g, be, batch, seq, wfc, bfc, wlin, blin)


@jax.jit
def kernel(embedding_features_per_residue, edge_index, batch,
           embedding_features_per_sequence,
           W1, b1, g1, be1, W2, b2, g2, be2, W3, b3, g3, be3,
           Wfc, bfc, Wlin, blin):
  x = embedding_features_per_residue
  src = edge_index[0]
  dst = edge_index[1]
  zrows = jnp.zeros((NPAD, 128), jnp.float32)
  zvec = jnp.zeros((NPAD,), jnp.float32)

  deg_call, scatter_call = _sc_calls()
  degp = deg_call(dst, zvec)
  degp = degp[:, :N, None]

  b1r, g1r, be1r = b1[None], g1[None], be1[None]
  b2r, g2r, be2r = b2[None], g2[None], be2[None]
  b3r, g3r, be3r = b3[None], g3[None], be3[None]

  hp1 = _pre_call(degp, x, W1)
  acc1 = scatter_call(src, dst, hp1, zrows)[:, :N]
  hp2 = _mid_call(degp, acc1, hp1, b1r, g1r, be1r, W2, relu=True)
  acc2 = scatter_call(src, dst, hp2, zrows)[:, :N]
  hp3 = _mid_call(degp, acc2, hp2, b2r, g2r, be2r, W3, relu=True)
  acc3 = scatter_call(src, dst, hp3, zrows)[:, :N]
  return _head_call(degp, acc3, hp3, b3r, g3r, be3r, batch[None],
                    embedding_features_per_sequence, Wfc, bfc[None],
                    Wlin, blin[None])


# speedup vs baseline: 25.5768x; 1.1167x over previous
import functools

import jax
import jax.numpy as jnp
from jax import lax
from jax.experimental import pallas as pl
from jax.experimental.pallas import tpu as pltpu
from jax.experimental.pallas import tpu_sc as plsc

N = 10000
E = 320000
NPAD = 10240
SLAB = NPAD // 16
NTILES = 32
ET = E // NTILES
CH = 80
NCH = ET // CH


def _deg_body(dst_hbm, zvec_hbm, out_hbm, didx, ones_v, deg_sh):
  c = lax.axis_index("c")
  s = lax.axis_index("s")
  tid = s * 2 + c
  base = tid * ET

  pltpu.sync_copy(zvec_hbm.at[pl.ds(s * SLAB, SLAB)],
                  deg_sh.at[pl.ds(s * SLAB, SLAB)])
  for i in range(CH // 16):
    ones_v[pl.ds(i * 16, 16)] = jnp.ones((16,), jnp.float32)
  plsc.subcore_barrier()

  def step(k, carry):
    pltpu.sync_copy(dst_hbm.at[pl.ds(base + k * CH, CH)], didx)
    pltpu.sync_copy(ones_v, deg_sh.at[didx], add=True)
    return carry

  lax.fori_loop(0, NCH, step, 0)
  plsc.subcore_barrier()
  pltpu.sync_copy(deg_sh.at[pl.ds(s * SLAB, SLAB)],
                  out_hbm.at[c, pl.ds(s * SLAB, SLAB)])


def _scatter_body(src_hbm, dst_hbm, hp_hbm, zrows_hbm, out_hbm,
                  sidx_all, didx0, didx1, didx2, rows0, rows1, rows2, acc_sh,
                  gsem0, gsem1, gsem2, dsem0, dsem1, dsem2,
                  ssem0, ssem1, ssem2):
  c = lax.axis_index("c")
  s = lax.axis_index("s")
  tid = s * 2 + c
  base = tid * ET

  pltpu.sync_copy(zrows_hbm.at[pl.ds(s * SLAB, SLAB)],
                  acc_sh.at[pl.ds(s * SLAB, SLAB)])
  pltpu.sync_copy(src_hbm.at[pl.ds(base, ET)], sidx_all)
  plsc.subcore_barrier()

  didx = (didx0, didx1, didx2)
  rows = (rows0, rows1, rows2)
  gsem = (gsem0, gsem1, gsem2)
  dsem = (dsem0, dsem1, dsem2)
  ssem = (ssem0, ssem1, ssem2)

  def prefetch(i, b):
    pltpu.async_copy(dst_hbm.at[pl.ds(base + i * CH, CH)], didx[b], dsem[b])
    pltpu.async_copy(hp_hbm.at[sidx_all.at[pl.ds(i * CH, CH)]], rows[b],
                     gsem[b])

  def wait_inputs(b):
    pltpu.make_async_copy(dst_hbm.at[pl.ds(base, CH)], didx[b],
                          dsem[b]).wait()
    pltpu.make_async_copy(hp_hbm.at[sidx_all.at[pl.ds(0, CH)]], rows[b],
                          gsem[b]).wait()

  def wait_scatter(b):
    pltpu.make_async_copy(rows[b], acc_sh.at[didx[b]], ssem[b]).wait()

  def scatter(i, b):
    pltpu.async_copy(rows[b], acc_sh.at[didx[b]], ssem[b], add=True)

  prefetch(0, 0)
  prefetch(1, 1)
  wait_inputs(0); scatter(0, 0); prefetch(2, 2)
  wait_inputs(1); scatter(1, 1); wait_scatter(0); prefetch(3, 0)
  wait_inputs(2); scatter(2, 2); wait_scatter(1); prefetch(4, 1)

  def outer(j, carry):
    for b in range(3):
      i = j * 3 + b
      wait_inputs(b)
      scatter(i, b)
      sb = (b + 2) % 3
      wait_scatter(sb)
      @pl.when(i + 2 < NCH)
      def _():
        prefetch(i + 2, sb)
    return carry

  lax.fori_loop(1, NCH // 3, outer, 0)
  for i in range(3 * (NCH // 3), NCH):
    b = i % 3
    wait_inputs(b)
    scatter(i, b)
    wait_scatter((b + 2) % 3)
  wait_scatter((NCH - 1) % 3)

  plsc.subcore_barrier()
  pltpu.sync_copy(acc_sh.at[pl.ds(s * SLAB, SLAB)],
                  out_hbm.at[c, pl.ds(s * SLAB, SLAB)])


@functools.lru_cache(maxsize=None)
def _sc_calls():
  mesh = plsc.VectorSubcoreMesh(core_axis_name="c", subcore_axis_name="s")
  deg_call = pl.kernel(
      _deg_body,
      out_type=jax.ShapeDtypeStruct((2, NPAD), jnp.float32),
      mesh=mesh,
      scratch_types=[
          pltpu.VMEM((CH,), jnp.int32),
          pltpu.VMEM((CH,), jnp.float32),
          pltpu.VMEM_SHARED((NPAD,), jnp.float32),
      ],
  )
  scatter_call = pl.kernel(
      _scatter_body,
      out_type=jax.ShapeDtypeStruct((2, NPAD, 128), jnp.float32),
      mesh=mesh,
      scratch_types=[
          pltpu.VMEM((ET,), jnp.int32),
          pltpu.VMEM((CH,), jnp.int32),
          pltpu.VMEM((CH,), jnp.int32),
          pltpu.VMEM((CH,), jnp.int32),
          pltpu.VMEM((CH, 128), jnp.float32),
          pltpu.VMEM((CH, 128), jnp.float32),
          pltpu.VMEM((CH, 128), jnp.float32),
          pltpu.VMEM_SHARED((NPAD, 128), jnp.float32),
      ] + [pltpu.SemaphoreType.DMA] * 9,
  )
  return deg_call, scatter_call


def _dinv(degp_ref):
  deg = degp_ref[0] + degp_ref[1] + 1.0
  return lax.rsqrt(deg)


def _pre_body(degp_ref, x_ref, w_ref, hp_ref):
  h = jnp.dot(x_ref[...], w_ref[...], preferred_element_type=jnp.float32)
  hp_ref[...] = h * _dinv(degp_ref)


def _pre_call(degp, x, w):
  return pl.pallas_call(
      _pre_body,
      out_shape=jax.ShapeDtypeStruct((N, 128), jnp.float32),
  )(degp, x, w)


def _bn(z, g_ref, be_ref):
  mean = jnp.mean(z, axis=0, keepdims=True)
  d = z - mean
  var = jnp.mean(d * d, axis=0, keepdims=True)
  return d * lax.rsqrt(var + 1e-5) * g_ref[...] + be_ref[...]


def _mid_body(degp_ref, acc_ref, hp_ref, b_ref, g_ref, be_ref, w_ref,
              out_ref, *, relu):
  dinv = _dinv(degp_ref)
  z = dinv * (acc_ref[0] + acc_ref[1] + hp_ref[...]) + b_ref[...]
  if relu:
    z = jnp.maximum(z, 0.0)
  z = _bn(z, g_ref, be_ref)
  out_ref[...] = jnp.dot(z, w_ref[...],
                         preferred_element_type=jnp.float32) * dinv


def _mid_call(degp, acc, hp, b, g, be, w, relu):
  return pl.pallas_call(
      functools.partial(_mid_body, relu=relu),
      out_shape=jax.ShapeDtypeStruct((N, 128), jnp.float32),
  )(degp, acc, hp, b, g, be, w)


def _head_body(degp_ref, acc_ref, hp_ref, b_ref, g_ref, be_ref,
               batch_ref, seq_ref, wfc_ref, bfc_ref, wlin_ref, blin_ref,
               out_ref):
  dinv = _dinv(degp_ref)
  z = dinv * (acc_ref[0] + acc_ref[1] + hp_ref[...]) + b_ref[...]
  z = _bn(z, g_ref, be_ref)
  gids = lax.broadcasted_iota(jnp.int32, (16, N), 0)
  mask = (gids == batch_ref[...]).astype(jnp.float32)
  cnt = jnp.sum(mask, axis=1, keepdims=True)
  pooled = jnp.dot(mask, z, preferred_element_type=jnp.float32)
  pooled = pooled / jnp.maximum(cnt, 1.0)
  y = jnp.dot(seq_ref[...], wfc_ref[...],
              preferred_element_type=jnp.float32) + bfc_ref[...]
  o = jnp.dot(pooled + y, wlin_ref[...],
              preferred_element_type=jnp.float32) + blin_ref[...]
  out_ref[...] = jax.nn.sigmoid(o)


def _head_call(degp, acc, hp, b, g, be, batch, seq, wfc, bfc, wlin, blin):
  return pl.pallas_call(
      _head_body,
      out_shape=jax.ShapeDtypeStruct((16, 256), jnp.float32),
  )(degp, acc, hp, b, g, be, batch, seq, wfc, bfc, wlin, blin)


@jax.jit
def kernel(embedding_features_per_residue, edge_index, batch,
           embedding_features_per_sequence,
           W1, b1, g1, be1, W2, b2, g2, be2, W3, b3, g3, be3,
           Wfc, bfc, Wlin, blin):
  x = embedding_features_per_residue
  src = edge_index[0]
  dst = edge_index[1]
  zrows = jnp.zeros((NPAD, 128), jnp.float32)
  zvec = jnp.zeros((NPAD,), jnp.float32)

  deg_call, scatter_call = _sc_calls()
  degp = deg_call(dst, zvec)
  degp = degp[:, :N, None]

  b1r, g1r, be1r = b1[None], g1[None], be1[None]
  b2r, g2r, be2r = b2[None], g2[None], be2[None]
  b3r, g3r, be3r = b3[None], g3[None], be3[None]

  hp1 = _pre_call(degp, x, W1)
  acc1 = scatter_call(src, dst, hp1, zrows)[:, :N]
  hp2 = _mid_call(degp, acc1, hp1, b1r, g1r, be1r, W2, relu=True)
  acc2 = scatter_call(src, dst, hp2, zrows)[:, :N]
  hp3 = _mid_call(degp, acc2, hp2, b2r, g2r, be2r, W3, relu=True)
  acc3 = scatter_call(src, dst, hp3, zrows)[:, :N]
  return _head_call(degp, acc3, hp3, b3r, g3r, be3r, batch[None],
                    embedding_features_per_sequence, Wfc, bfc[None],
                    Wlin, blin[None])


# speedup vs baseline: 28.9993x; 1.1338x over previous
import functools

import jax
import jax.numpy as jnp
from jax import lax
from jax.experimental import pallas as pl
from jax.experimental.pallas import tpu as pltpu
from jax.experimental.pallas import tpu_sc as plsc

N = 10000
E = 320000
NPAD = 10240
SLAB = NPAD // 16
NTILES = 32
ET = E // NTILES
CH = 80
NCH = ET // CH


def _deg_body(dst_hbm, zvec_hbm, out_hbm, didx0, didx1, didx2, ones_v,
              deg_sh, dsem0, dsem1, dsem2, ssem0, ssem1, ssem2):
  c = lax.axis_index("c")
  s = lax.axis_index("s")
  tid = s * 2 + c
  base = tid * ET

  pltpu.sync_copy(zvec_hbm.at[pl.ds(s * SLAB, SLAB)],
                  deg_sh.at[pl.ds(s * SLAB, SLAB)])
  for i in range(CH // 16):
    ones_v[pl.ds(i * 16, 16)] = jnp.ones((16,), jnp.float32)
  plsc.subcore_barrier()

  didx = (didx0, didx1, didx2)
  dsem = (dsem0, dsem1, dsem2)
  ssem = (ssem0, ssem1, ssem2)

  def prefetch(i, b):
    pltpu.async_copy(dst_hbm.at[pl.ds(base + i * CH, CH)], didx[b], dsem[b])

  def wait_idx(b):
    pltpu.make_async_copy(dst_hbm.at[pl.ds(base, CH)], didx[b],
                          dsem[b]).wait()

  def scatter(b):
    pltpu.async_copy(ones_v, deg_sh.at[didx[b]], ssem[b], add=True)

  def wait_scatter(b):
    pltpu.make_async_copy(ones_v, deg_sh.at[didx[b]], ssem[b]).wait()

  prefetch(0, 0)
  prefetch(1, 1)
  wait_idx(0); scatter(0); prefetch(2, 2)
  wait_idx(1); scatter(1); wait_scatter(0); prefetch(3, 0)
  wait_idx(2); scatter(2); wait_scatter(1); prefetch(4, 1)

  def outer(j, carry):
    for b in range(3):
      i = j * 3 + b
      wait_idx(b)
      scatter(b)
      sb = (b + 2) % 3
      wait_scatter(sb)

      @pl.when(i + 2 < NCH)
      def _():
        prefetch(i + 2, sb)
    return carry

  lax.fori_loop(1, NCH // 3, outer, 0)
  for i in range(3 * (NCH // 3), NCH):
    b = i % 3
    wait_idx(b)
    scatter(b)
    wait_scatter((b + 2) % 3)
  wait_scatter((NCH - 1) % 3)

  plsc.subcore_barrier()
  pltpu.sync_copy(deg_sh.at[pl.ds(s * SLAB, SLAB)],
                  out_hbm.at[c, pl.ds(s * SLAB, SLAB)])


def _scatter_body(src_hbm, dst_hbm, hp_hbm, zrows_hbm, out_hbm,
                  sidx_all, didx0, didx1, didx2, rows0, rows1, rows2, acc_sh,
                  gsem0, gsem1, gsem2, dsem0, dsem1, dsem2,
                  ssem0, ssem1, ssem2):
  c = lax.axis_index("c")
  s = lax.axis_index("s")
  tid = s * 2 + c
  base = tid * ET

  pltpu.sync_copy(zrows_hbm.at[pl.ds(s * SLAB, SLAB)],
                  acc_sh.at[pl.ds(s * SLAB, SLAB)])
  pltpu.sync_copy(src_hbm.at[pl.ds(base, ET)], sidx_all)
  plsc.subcore_barrier()

  didx = (didx0, didx1, didx2)
  rows = (rows0, rows1, rows2)
  gsem = (gsem0, gsem1, gsem2)
  dsem = (dsem0, dsem1, dsem2)
  ssem = (ssem0, ssem1, ssem2)

  def prefetch(i, b):
    pltpu.async_copy(dst_hbm.at[pl.ds(base + i * CH, CH)], didx[b], dsem[b])
    pltpu.async_copy(hp_hbm.at[sidx_all.at[pl.ds(i * CH, CH)]], rows[b],
                     gsem[b])

  def wait_inputs(b):
    pltpu.make_async_copy(dst_hbm.at[pl.ds(base, CH)], didx[b],
                          dsem[b]).wait()
    pltpu.make_async_copy(hp_hbm.at[sidx_all.at[pl.ds(0, CH)]], rows[b],
                          gsem[b]).wait()

  def wait_scatter(b):
    pltpu.make_async_copy(rows[b], acc_sh.at[didx[b]], ssem[b]).wait()

  def scatter(i, b):
    pltpu.async_copy(rows[b], acc_sh.at[didx[b]], ssem[b], add=True)

  prefetch(0, 0)
  prefetch(1, 1)
  wait_inputs(0); scatter(0, 0); prefetch(2, 2)
  wait_inputs(1); scatter(1, 1); wait_scatter(0); prefetch(3, 0)
  wait_inputs(2); scatter(2, 2); wait_scatter(1); prefetch(4, 1)

  def outer(j, carry):
    for b in range(3):
      i = j * 3 + b
      wait_inputs(b)
      scatter(i, b)
      sb = (b + 2) % 3
      wait_scatter(sb)
      @pl.when(i + 2 < NCH)
      def _():
        prefetch(i + 2, sb)
    return carry

  lax.fori_loop(1, NCH // 3, outer, 0)
  for i in range(3 * (NCH // 3), NCH):
    b = i % 3
    wait_inputs(b)
    scatter(i, b)
    wait_scatter((b + 2) % 3)
  wait_scatter((NCH - 1) % 3)

  plsc.subcore_barrier()
  pltpu.sync_copy(acc_sh.at[pl.ds(s * SLAB, SLAB)],
                  out_hbm.at[c, pl.ds(s * SLAB, SLAB)])


@functools.lru_cache(maxsize=None)
def _sc_calls():
  mesh = plsc.VectorSubcoreMesh(core_axis_name="c", subcore_axis_name="s")
  deg_call = pl.kernel(
      _deg_body,
      out_type=jax.ShapeDtypeStruct((2, NPAD), jnp.float32),
      mesh=mesh,
      scratch_types=[
          pltpu.VMEM((CH,), jnp.int32),
          pltpu.VMEM((CH,), jnp.int32),
          pltpu.VMEM((CH,), jnp.int32),
          pltpu.VMEM((CH,), jnp.float32),
          pltpu.VMEM_SHARED((NPAD,), jnp.float32),
      ] + [pltpu.SemaphoreType.DMA] * 6,
  )
  scatter_call = pl.kernel(
      _scatter_body,
      out_type=jax.ShapeDtypeStruct((2, NPAD, 128), jnp.float32),
      mesh=mesh,
      scratch_types=[
          pltpu.VMEM((ET,), jnp.int32),
          pltpu.VMEM((CH,), jnp.int32),
          pltpu.VMEM((CH,), jnp.int32),
          pltpu.VMEM((CH,), jnp.int32),
          pltpu.VMEM((CH, 128), jnp.float32),
          pltpu.VMEM((CH, 128), jnp.float32),
          pltpu.VMEM((CH, 128), jnp.float32),
          pltpu.VMEM_SHARED((NPAD, 128), jnp.float32),
      ] + [pltpu.SemaphoreType.DMA] * 9,
  )
  return deg_call, scatter_call


def _dinv(degp_ref):
  deg = degp_ref[0] + degp_ref[1] + 1.0
  return lax.rsqrt(deg)


def _pre_body(degp_ref, x_ref, w_ref, hp_ref):
  h = jnp.dot(x_ref[...], w_ref[...], preferred_element_type=jnp.float32)
  hp_ref[...] = h * _dinv(degp_ref)


def _pre_call(degp, x, w):
  return pl.pallas_call(
      _pre_body,
      out_shape=jax.ShapeDtypeStruct((N, 128), jnp.float32),
  )(degp, x, w)


def _bn(z, g_ref, be_ref):
  mean = jnp.mean(z, axis=0, keepdims=True)
  d = z - mean
  var = jnp.mean(d * d, axis=0, keepdims=True)
  return d * lax.rsqrt(var + 1e-5) * g_ref[...] + be_ref[...]


def _mid_body(degp_ref, acc_ref, hp_ref, b_ref, g_ref, be_ref, w_ref,
              out_ref, *, relu):
  dinv = _dinv(degp_ref)
  z = dinv * (acc_ref[0, :N] + acc_ref[1, :N] + hp_ref[...]) + b_ref[...]
  if relu:
    z = jnp.maximum(z, 0.0)
  z = _bn(z, g_ref, be_ref)
  out_ref[...] = jnp.dot(z, w_ref[...],
                         preferred_element_type=jnp.float32) * dinv


def _mid_call(degp, acc, hp, b, g, be, w, relu):
  return pl.pallas_call(
      functools.partial(_mid_body, relu=relu),
      out_shape=jax.ShapeDtypeStruct((N, 128), jnp.float32),
  )(degp, acc, hp, b, g, be, w)


def _head_body(degp_ref, acc_ref, hp_ref, b_ref, g_ref, be_ref,
               batch_ref, seq_ref, wfc_ref, bfc_ref, wlin_ref, blin_ref,
               out_ref):
  dinv = _dinv(degp_ref)
  z = dinv * (acc_ref[0, :N] + acc_ref[1, :N] + hp_ref[...]) + b_ref[...]
  z = _bn(z, g_ref, be_ref)
  gids = lax.broadcasted_iota(jnp.int32, (16, N), 0)
  mask = (gids == batch_ref[...]).astype(jnp.float32)
  cnt = jnp.sum(mask, axis=1, keepdims=True)
  pooled = jnp.dot(mask, z, preferred_element_type=jnp.float32)
  pooled = pooled / jnp.maximum(cnt, 1.0)
  y = jnp.dot(seq_ref[...], wfc_ref[...],
              preferred_element_type=jnp.float32) + bfc_ref[...]
  o = jnp.dot(pooled + y, wlin_ref[...],
              preferred_element_type=jnp.float32) + blin_ref[...]
  out_ref[...] = jax.nn.sigmoid(o)


def _head_call(degp, acc, hp, b, g, be, batch, seq, wfc, bfc, wlin, blin):
  return pl.pallas_call(
      _head_body,
      out_shape=jax.ShapeDtypeStruct((16, 256), jnp.float32),
  )(degp, acc, hp, b, g, be, batch, seq, wfc, bfc, wlin, blin)


@jax.jit
def kernel(embedding_features_per_residue, edge_index, batch,
           embedding_features_per_sequence,
           W1, b1, g1, be1, W2, b2, g2, be2, W3, b3, g3, be3,
           Wfc, bfc, Wlin, blin):
  x = embedding_features_per_residue
  src = edge_index[0]
  dst = edge_index[1]
  zrows = jnp.zeros((NPAD, 128), jnp.float32)
  zvec = jnp.zeros((NPAD,), jnp.float32)

  deg_call, scatter_call = _sc_calls()
  degp = deg_call(dst, zvec)
  degp = degp[:, :N, None]

  b1r, g1r, be1r = b1[None], g1[None], be1[None]
  b2r, g2r, be2r = b2[None], g2[None], be2[None]
  b3r, g3r, be3r = b3[None], g3[None], be3[None]

  hp1 = _pre_call(degp, x, W1)
  acc1 = scatter_call(src, dst, hp1, zrows)
  hp2 = _mid_call(degp, acc1, hp1, b1r, g1r, be1r, W2, relu=True)
  acc2 = scatter_call(src, dst, hp2, zrows)
  hp3 = _mid_call(degp, acc2, hp2, b2r, g2r, be2r, W3, relu=True)
  acc3 = scatter_call(src, dst, hp3, zrows)
  return _head_call(degp, acc3, hp3, b3r, g3r, be3r, batch[None],
                    embedding_features_per_sequence, Wfc, bfc[None],
                    Wlin, blin[None])


# speedup vs baseline: 29.4151x; 1.0143x over previous
import functools

import jax
import jax.numpy as jnp
from jax import lax
from jax.experimental import pallas as pl
from jax.experimental.pallas import tpu as pltpu
from jax.experimental.pallas import tpu_sc as plsc

N = 10000
E = 320000
NPAD = 10240
SLAB = NPAD // 16
NTILES = 32
ET = E // NTILES
CH = 80
NCH = ET // CH


def _deg_body(dst_hbm, zvec_hbm, out_hbm, didx0, didx1, didx2, ones_v,
              deg_sh, dsem0, dsem1, dsem2, ssem0, ssem1, ssem2, zsem):
  c = lax.axis_index("c")
  s = lax.axis_index("s")
  tid = s * 2 + c
  base = tid * ET

  zcp = pltpu.async_copy(zvec_hbm.at[pl.ds(s * SLAB, SLAB)],
                         deg_sh.at[pl.ds(s * SLAB, SLAB)], zsem)
  for i in range(CH // 16):
    ones_v[pl.ds(i * 16, 16)] = jnp.ones((16,), jnp.float32)

  didx = (didx0, didx1, didx2)
  dsem = (dsem0, dsem1, dsem2)
  ssem = (ssem0, ssem1, ssem2)

  def prefetch(i, b):
    pltpu.async_copy(dst_hbm.at[pl.ds(base + i * CH, CH)], didx[b], dsem[b])

  def wait_idx(b):
    pltpu.make_async_copy(dst_hbm.at[pl.ds(base, CH)], didx[b],
                          dsem[b]).wait()

  def scatter(b):
    pltpu.async_copy(ones_v, deg_sh.at[didx[b]], ssem[b], add=True)

  def wait_scatter(b):
    pltpu.make_async_copy(ones_v, deg_sh.at[didx[b]], ssem[b]).wait()

  prefetch(0, 0)
  prefetch(1, 1)
  prefetch(2, 2)
  zcp.wait()
  plsc.subcore_barrier()
  wait_idx(0); scatter(0)
  wait_idx(1); scatter(1); wait_scatter(0); prefetch(3, 0)
  wait_idx(2); scatter(2); wait_scatter(1); prefetch(4, 1)

  def outer(j, carry):
    for b in range(3):
      i = j * 3 + b
      wait_idx(b)
      scatter(b)
      sb = (b + 2) % 3
      wait_scatter(sb)

      @pl.when(i + 2 < NCH)
      def _():
        prefetch(i + 2, sb)
    return carry

  lax.fori_loop(1, NCH // 3, outer, 0)
  for i in range(3 * (NCH // 3), NCH):
    b = i % 3
    wait_idx(b)
    scatter(b)
    wait_scatter((b + 2) % 3)
  wait_scatter((NCH - 1) % 3)

  plsc.subcore_barrier()
  pltpu.sync_copy(deg_sh.at[pl.ds(s * SLAB, SLAB)],
                  out_hbm.at[c, pl.ds(s * SLAB, SLAB)])


def _scatter_body(src_hbm, dst_hbm, hp_hbm, zrows_hbm, out_hbm,
                  sidx_all, didx0, didx1, didx2, rows0, rows1, rows2, acc_sh,
                  gsem0, gsem1, gsem2, dsem0, dsem1, dsem2,
                  ssem0, ssem1, ssem2, zsem):
  c = lax.axis_index("c")
  s = lax.axis_index("s")
  tid = s * 2 + c
  base = tid * ET

  zcp = pltpu.async_copy(zrows_hbm.at[pl.ds(s * SLAB, SLAB)],
                         acc_sh.at[pl.ds(s * SLAB, SLAB)], zsem)
  pltpu.sync_copy(src_hbm.at[pl.ds(base, ET)], sidx_all)

  didx = (didx0, didx1, didx2)
  rows = (rows0, rows1, rows2)
  gsem = (gsem0, gsem1, gsem2)
  dsem = (dsem0, dsem1, dsem2)
  ssem = (ssem0, ssem1, ssem2)

  def prefetch(i, b):
    pltpu.async_copy(dst_hbm.at[pl.ds(base + i * CH, CH)], didx[b], dsem[b])
    pltpu.async_copy(hp_hbm.at[sidx_all.at[pl.ds(i * CH, CH)]], rows[b],
                     gsem[b])

  def wait_inputs(b):
    pltpu.make_async_copy(dst_hbm.at[pl.ds(base, CH)], didx[b],
                          dsem[b]).wait()
    pltpu.make_async_copy(hp_hbm.at[sidx_all.at[pl.ds(0, CH)]], rows[b],
                          gsem[b]).wait()

  def wait_scatter(b):
    pltpu.make_async_copy(rows[b], acc_sh.at[didx[b]], ssem[b]).wait()

  def scatter(i, b):
    pltpu.async_copy(rows[b], acc_sh.at[didx[b]], ssem[b], add=True)

  prefetch(0, 0)
  prefetch(1, 1)
  prefetch(2, 2)
  zcp.wait()
  plsc.subcore_barrier()
  wait_inputs(0); scatter(0, 0)
  wait_inputs(1); scatter(1, 1); wait_scatter(0); prefetch(3, 0)
  wait_inputs(2); scatter(2, 2); wait_scatter(1); prefetch(4, 1)

  def outer(j, carry):
    for b in range(3):
      i = j * 3 + b
      wait_inputs(b)
      scatter(i, b)
      sb = (b + 2) % 3
      wait_scatter(sb)
      @pl.when(i + 2 < NCH)
      def _():
        prefetch(i + 2, sb)
    return carry

  lax.fori_loop(1, NCH // 3, outer, 0)
  for i in range(3 * (NCH // 3), NCH):
    b = i % 3
    wait_inputs(b)
    scatter(i, b)
    wait_scatter((b + 2) % 3)
  wait_scatter((NCH - 1) % 3)

  plsc.subcore_barrier()
  pltpu.sync_copy(acc_sh.at[pl.ds(s * SLAB, SLAB)],
                  out_hbm.at[c, pl.ds(s * SLAB, SLAB)])


@functools.lru_cache(maxsize=None)
def _sc_calls():
  mesh = plsc.VectorSubcoreMesh(core_axis_name="c", subcore_axis_name="s")
  deg_call = pl.kernel(
      _deg_body,
      out_type=jax.ShapeDtypeStruct((2, NPAD), jnp.float32),
      mesh=mesh,
      scratch_types=[
          pltpu.VMEM((CH,), jnp.int32),
          pltpu.VMEM((CH,), jnp.int32),
          pltpu.VMEM((CH,), jnp.int32),
          pltpu.VMEM((CH,), jnp.float32),
          pltpu.VMEM_SHARED((NPAD,), jnp.float32),
      ] + [pltpu.SemaphoreType.DMA] * 7,
  )
  scatter_call = pl.kernel(
      _scatter_body,
      out_type=jax.ShapeDtypeStruct((2, NPAD, 128), jnp.float32),
      mesh=mesh,
      scratch_types=[
          pltpu.VMEM((ET,), jnp.int32),
          pltpu.VMEM((CH,), jnp.int32),
          pltpu.VMEM((CH,), jnp.int32),
          pltpu.VMEM((CH,), jnp.int32),
          pltpu.VMEM((CH, 128), jnp.float32),
          pltpu.VMEM((CH, 128), jnp.float32),
          pltpu.VMEM((CH, 128), jnp.float32),
          pltpu.VMEM_SHARED((NPAD, 128), jnp.float32),
      ] + [pltpu.SemaphoreType.DMA] * 10,
  )
  return deg_call, scatter_call


def _dinv(degp_ref):
  deg = degp_ref[0] + degp_ref[1] + 1.0
  return lax.rsqrt(deg)


def _mm_body(x_ref, w_ref, h_ref):
  h_ref[...] = jnp.dot(x_ref[...], w_ref[...],
                       preferred_element_type=jnp.float32)


def _mm_call(x, w):
  return pl.pallas_call(
      _mm_body,
      out_shape=jax.ShapeDtypeStruct((N, 128), jnp.float32),
  )(x, w)


def _scale_body(degp_ref, h_ref, hp_ref):
  hp_ref[...] = h_ref[...] * _dinv(degp_ref)


def _scale_call(degp, h):
  return pl.pallas_call(
      _scale_body,
      out_shape=jax.ShapeDtypeStruct((N, 128), jnp.float32),
  )(degp, h)


def _bn(z, g_ref, be_ref):
  mean = jnp.mean(z, axis=0, keepdims=True)
  d = z - mean
  var = jnp.mean(d * d, axis=0, keepdims=True)
  return d * lax.rsqrt(var + 1e-5) * g_ref[...] + be_ref[...]


def _mid_body(degp_ref, acc_ref, hp_ref, b_ref, g_ref, be_ref, w_ref,
              out_ref, *, relu):
  dinv = _dinv(degp_ref)
  z = dinv * (acc_ref[0, :N] + acc_ref[1, :N] + hp_ref[...]) + b_ref[...]
  if relu:
    z = jnp.maximum(z, 0.0)
  z = _bn(z, g_ref, be_ref)
  out_ref[...] = jnp.dot(z, w_ref[...],
                         preferred_element_type=jnp.float32) * dinv


def _mid_call(degp, acc, hp, b, g, be, w, relu):
  return pl.pallas_call(
      functools.partial(_mid_body, relu=relu),
      out_shape=jax.ShapeDtypeStruct((N, 128), jnp.float32),
  )(degp, acc, hp, b, g, be, w)


def _head_body(degp_ref, acc_ref, hp_ref, b_ref, g_ref, be_ref,
               batch_ref, seq_ref, wfc_ref, bfc_ref, wlin_ref, blin_ref,
               out_ref):
  dinv = _dinv(degp_ref)
  z = dinv * (acc_ref[0, :N] + acc_ref[1, :N] + hp_ref[...]) + b_ref[...]
  z = _bn(z, g_ref, be_ref)
  gids = lax.broadcasted_iota(jnp.int32, (16, N), 0)
  mask = (gids == batch_ref[...]).astype(jnp.float32)
  cnt = jnp.sum(mask, axis=1, keepdims=True)
  pooled = jnp.dot(mask, z, preferred_element_type=jnp.float32)
  pooled = pooled / jnp.maximum(cnt, 1.0)
  y = jnp.dot(seq_ref[...], wfc_ref[...],
              preferred_element_type=jnp.float32) + bfc_ref[...]
  o = jnp.dot(pooled + y, wlin_ref[...],
              preferred_element_type=jnp.float32) + blin_ref[...]
  out_ref[...] = jax.nn.sigmoid(o)


def _head_call(degp, acc, hp, b, g, be, batch, seq, wfc, bfc, wlin, blin):
  return pl.pallas_call(
      _head_body,
      out_shape=jax.ShapeDtypeStruct((16, 256), jnp.float32),
  )(degp, acc, hp, b, g, be, batch, seq, wfc, bfc, wlin, blin)


@jax.jit
def kernel(embedding_features_per_residue, edge_index, batch,
           embedding_features_per_sequence,
           W1, b1, g1, be1, W2, b2, g2, be2, W3, b3, g3, be3,
           Wfc, bfc, Wlin, blin):
  x = embedding_features_per_residue
  src = edge_index[0]
  dst = edge_index[1]
  zrows = jnp.zeros((NPAD, 128), jnp.float32)
  zvec = jnp.zeros((NPAD,), jnp.float32)

  deg_call, scatter_call = _sc_calls()
  degp = deg_call(dst, zvec)
  degp = degp[:, :N, None]

  b1r, g1r, be1r = b1[None], g1[None], be1[None]
  b2r, g2r, be2r = b2[None], g2[None], be2[None]
  b3r, g3r, be3r = b3[None], g3[None], be3[None]

  h1 = _mm_call(x, W1)
  hp1 = _scale_call(degp, h1)
  acc1 = scatter_call(src, dst, hp1, zrows)
  hp2 = _mid_call(degp, acc1, hp1, b1r, g1r, be1r, W2, relu=True)
  acc2 = scatter_call(src, dst, hp2, zrows)
  hp3 = _mid_call(degp, acc2, hp2, b2r, g2r, be2r, W3, relu=True)
  acc3 = scatter_call(src, dst, hp3, zrows)
  return _head_call(degp, acc3, hp3, b3r, g3r, be3r, batch[None],
                    embedding_features_per_sequence, Wfc, bfc[None],
                    Wlin, blin[None])


# speedup vs baseline: 29.4749x; 1.0020x over previous
import functools

import jax
import jax.numpy as jnp
from jax import lax
from jax.experimental import pallas as pl
from jax.experimental.pallas import tpu as pltpu
from jax.experimental.pallas import tpu_sc as plsc

N = 10000
E = 320000
NPAD = 10240
SLAB = NPAD // 16
NTILES = 32
ET = E // NTILES
CH = 80
NCH = ET // CH


def _deg_body(dst_hbm, zvec_hbm, out_hbm, didx0, didx1, didx2, ones_v,
              deg_sh, dsem0, dsem1, dsem2, ssem0, ssem1, ssem2, zsem):
  c = lax.axis_index("c")
  s = lax.axis_index("s")
  tid = s * 2 + c
  base = tid * ET

  zcp = pltpu.async_copy(zvec_hbm.at[pl.ds(s * SLAB, SLAB)],
                         deg_sh.at[pl.ds(s * SLAB, SLAB)], zsem)
  for i in range(CH // 16):
    ones_v[pl.ds(i * 16, 16)] = jnp.ones((16,), jnp.float32)

  didx = (didx0, didx1, didx2)
  dsem = (dsem0, dsem1, dsem2)
  ssem = (ssem0, ssem1, ssem2)

  def prefetch(i, b):
    pltpu.async_copy(dst_hbm.at[pl.ds(base + i * CH, CH)], didx[b], dsem[b])

  def wait_idx(b):
    pltpu.make_async_copy(dst_hbm.at[pl.ds(base, CH)], didx[b],
                          dsem[b]).wait()

  def scatter(b):
    pltpu.async_copy(ones_v, deg_sh.at[didx[b]], ssem[b], add=True)

  def wait_scatter(b):
    pltpu.make_async_copy(ones_v, deg_sh.at[didx[b]], ssem[b]).wait()

  prefetch(0, 0)
  prefetch(1, 1)
  prefetch(2, 2)
  zcp.wait()
  plsc.subcore_barrier()
  wait_idx(0); scatter(0)
  wait_idx(1); scatter(1); wait_scatter(0); prefetch(3, 0)
  wait_idx(2); scatter(2); wait_scatter(1); prefetch(4, 1)

  def outer(j, carry):
    for b in range(3):
      i = j * 3 + b
      wait_idx(b)
      scatter(b)
      sb = (b + 2) % 3
      wait_scatter(sb)

      @pl.when(i + 2 < NCH)
      def _():
        prefetch(i + 2, sb)
    return carry

  lax.fori_loop(1, NCH // 3, outer, 0)
  for i in range(3 * (NCH // 3), NCH):
    b = i % 3
    wait_idx(b)
    scatter(b)
    wait_scatter((b + 2) % 3)
  wait_scatter((NCH - 1) % 3)

  plsc.subcore_barrier()
  pltpu.sync_copy(deg_sh.at[pl.ds(s * SLAB, SLAB)],
                  out_hbm.at[c, pl.ds(s * SLAB, SLAB)])


def _scatter_body(src_hbm, dst_hbm, hp_hbm, zrows_hbm, out_hbm,
                  sidx_all, didx0, didx1, didx2, rows0, rows1, rows2, acc_sh,
                  gsem0, gsem1, gsem2, dsem0, dsem1, dsem2,
                  ssem0, ssem1, ssem2, zsem):
  c = lax.axis_index("c")
  s = lax.axis_index("s")
  tid = s * 2 + c
  base = tid * ET

  zcp = pltpu.async_copy(zrows_hbm.at[pl.ds(s * SLAB, SLAB)],
                         acc_sh.at[pl.ds(s * SLAB, SLAB)], zsem)
  pltpu.sync_copy(src_hbm.at[pl.ds(base, ET)], sidx_all)

  didx = (didx0, didx1, didx2)
  rows = (rows0, rows1, rows2)
  gsem = (gsem0, gsem1, gsem2)
  dsem = (dsem0, dsem1, dsem2)
  ssem = (ssem0, ssem1, ssem2)

  def prefetch(i, b):
    pltpu.async_copy(dst_hbm.at[pl.ds(base + i * CH, CH)], didx[b], dsem[b])
    pltpu.async_copy(hp_hbm.at[sidx_all.at[pl.ds(i * CH, CH)]], rows[b],
                     gsem[b])

  def wait_inputs(b):
    pltpu.make_async_copy(dst_hbm.at[pl.ds(base, CH)], didx[b],
                          dsem[b]).wait()
    pltpu.make_async_copy(hp_hbm.at[sidx_all.at[pl.ds(0, CH)]], rows[b],
                          gsem[b]).wait()

  def wait_scatter(b):
    pltpu.make_async_copy(rows[b], acc_sh.at[didx[b]], ssem[b]).wait()

  def scatter(i, b):
    pltpu.async_copy(rows[b], acc_sh.at[didx[b]], ssem[b], add=True)

  prefetch(0, 0)
  prefetch(1, 1)
  prefetch(2, 2)
  zcp.wait()
  plsc.subcore_barrier()
  wait_inputs(0); scatter(0, 0)
  wait_inputs(1); scatter(1, 1); wait_scatter(0); prefetch(3, 0)
  wait_inputs(2); scatter(2, 2); wait_scatter(1); prefetch(4, 1)

  def outer(j, carry):
    for b in range(3):
      i = j * 3 + b
      wait_inputs(b)
      scatter(i, b)
      sb = (b + 2) % 3
      wait_scatter(sb)
      @pl.when(i + 2 < NCH)
      def _():
        prefetch(i + 2, sb)
    return carry

  lax.fori_loop(1, NCH // 3, outer, 0)
  for i in range(3 * (NCH // 3), NCH):
    b = i % 3
    wait_inputs(b)
    scatter(i, b)
    wait_scatter((b + 2) % 3)
  wait_scatter((NCH - 1) % 3)

  plsc.subcore_barrier()
  pltpu.sync_copy(acc_sh.at[pl.ds(s * SLAB, SLAB)],
                  out_hbm.at[c, pl.ds(s * SLAB, SLAB)])


@functools.lru_cache(maxsize=None)
def _sc_calls():
  mesh = plsc.VectorSubcoreMesh(core_axis_name="c", subcore_axis_name="s")
  deg_call = pl.kernel(
      _deg_body,
      out_type=jax.ShapeDtypeStruct((2, NPAD), jnp.float32),
      mesh=mesh,
      scratch_types=[
          pltpu.VMEM((CH,), jnp.int32),
          pltpu.VMEM((CH,), jnp.int32),
          pltpu.VMEM((CH,), jnp.int32),
          pltpu.VMEM((CH,), jnp.float32),
          pltpu.VMEM_SHARED((NPAD,), jnp.float32),
      ] + [pltpu.SemaphoreType.DMA] * 7,
  )
  scatter_call = pl.kernel(
      _scatter_body,
      out_type=jax.ShapeDtypeStruct((2, NPAD, 128), jnp.float32),
      mesh=mesh,
      scratch_types=[
          pltpu.VMEM((ET,), jnp.int32),
          pltpu.VMEM((CH,), jnp.int32),
          pltpu.VMEM((CH,), jnp.int32),
          pltpu.VMEM((CH,), jnp.int32),
          pltpu.VMEM((CH, 128), jnp.float32),
          pltpu.VMEM((CH, 128), jnp.float32),
          pltpu.VMEM((CH, 128), jnp.float32),
          pltpu.VMEM_SHARED((NPAD, 128), jnp.float32),
      ] + [pltpu.SemaphoreType.DMA] * 10,
  )
  return deg_call, scatter_call


def _dinv(degp_ref):
  deg = degp_ref[0] + degp_ref[1] + 1.0
  return lax.rsqrt(deg)


def _pre_body(degp_ref, x_ref, w_ref, hp_ref):
  h = jnp.dot(x_ref[...], w_ref[...], preferred_element_type=jnp.float32)
  hp_ref[...] = h * _dinv(degp_ref)


def _pre_call(degp, x, w):
  return pl.pallas_call(
      _pre_body,
      out_shape=jax.ShapeDtypeStruct((N, 128), jnp.float32),
  )(degp, x, w)


def _bn(z, g_ref, be_ref):
  mean = jnp.mean(z, axis=0, keepdims=True)
  d = z - mean
  var = jnp.mean(d * d, axis=0, keepdims=True)
  return d * lax.rsqrt(var + 1e-5) * g_ref[...] + be_ref[...]


def _mid_body(degp_ref, acc_ref, hp_ref, b_ref, g_ref, be_ref, w_ref,
              out_ref, *, relu):
  dinv = _dinv(degp_ref)
  z = dinv * (acc_ref[0, :N] + acc_ref[1, :N] + hp_ref[...]) + b_ref[...]
  if relu:
    z = jnp.maximum(z, 0.0)
  z = _bn(z, g_ref, be_ref)
  out_ref[...] = jnp.dot(z, w_ref[...],
                         preferred_element_type=jnp.float32) * dinv


def _mid_call(degp, acc, hp, b, g, be, w, relu):
  return pl.pallas_call(
      functools.partial(_mid_body, relu=relu),
      out_shape=jax.ShapeDtypeStruct((N, 128), jnp.float32),
  )(degp, acc, hp, b, g, be, w)


def _head_body(degp_ref, acc_ref, hp_ref, b_ref, g_ref, be_ref,
               batch_ref, seq_ref, wfc_ref, bfc_ref, wlin_ref, blin_ref,
               out_ref):
  dinv = _dinv(degp_ref)
  z = dinv * (acc_ref[0, :N] + acc_ref[1, :N] + hp_ref[...]) + b_ref[...]
  z = _bn(z, g_ref, be_ref)
  gids = lax.broadcasted_iota(jnp.int32, (16, N), 0)
  mask = (gids == batch_ref[...]).astype(jnp.float32)
  cnt = jnp.sum(mask, axis=1, keepdims=True)
  pooled = jnp.dot(mask, z, preferred_element_type=jnp.float32)
  pooled = pooled / jnp.maximum(cnt, 1.0)
  y = jnp.dot(seq_ref[...], wfc_ref[...],
              preferred_element_type=jnp.float32) + bfc_ref[...]
  o = jnp.dot(pooled + y, wlin_ref[...],
              preferred_element_type=jnp.float32) + blin_ref[...]
  out_ref[...] = jax.nn.sigmoid(o)


def _head_call(degp, acc, hp, b, g, be, batch, seq, wfc, bfc, wlin, blin):
  return pl.pallas_call(
      _head_body,
      out_shape=jax.ShapeDtypeStruct((16, 256), jnp.float32),
  )(degp, acc, hp, b, g, be, batch, seq, wfc, bfc, wlin, blin)


@jax.jit
def kernel(embedding_features_per_residue, edge_index, batch,
           embedding_features_per_sequence,
           W1, b1, g1, be1, W2, b2, g2, be2, W3, b3, g3, be3,
           Wfc, bfc, Wlin, blin):
  x = embedding_features_per_residue
  src = edge_index[0]
  dst = edge_index[1]
  zrows = jnp.zeros((NPAD, 128), jnp.float32)
  zvec = jnp.zeros((NPAD,), jnp.float32)

  deg_call, scatter_call = _sc_calls()
  degp = deg_call(dst, zvec)
  degp = degp[:, :N, None]

  b1r, g1r, be1r = b1[None], g1[None], be1[None]
  b2r, g2r, be2r = b2[None], g2[None], be2[None]
  b3r, g3r, be3r = b3[None], g3[None], be3[None]

  hp1 = _pre_call(degp, x, W1)
  acc1 = scatter_call(src, dst, hp1, zrows)
  hp2 = _mid_call(degp, acc1, hp1, b1r, g1r, be1r, W2, relu=True)
  acc2 = scatter_call(src, dst, hp2, zrows)
  hp3 = _mid_call(degp, acc2, hp2, b2r, g2r, be2r, W3, relu=True)
  acc3 = scatter_call(src, dst, hp3, zrows)
  return _head_call(degp, acc3, hp3, b3r, g3r, be3r, batch[None],
                    embedding_features_per_sequence, Wfc, bfc[None],
                    Wlin, blin[None])
